# baseline (speedup 1.0000x reference)
import jax
import jax.numpy as jnp
from jax import lax
from jax.experimental import pallas as pl
from jax.experimental.pallas import tpu as pltpu
from jax.experimental.pallas import tpu_sc as plsc

N_CELLS = 100000
N_EDGES = 64
B_TOTAL = 100000
NC = 2
NS = 16
L = 16
NW = NC * NS
CH = 128
NG = CH // L
NCHUNK_G = -(-B_TOTAL // CH)
TAIL_BASE = B_TOTAL - CH
KPW = -(-NCHUNK_G // NW)

LN2 = 0.6931471805599453
HALF_LN2PI = 0.9189385332046727


def _fast_log(x):
    bits = plsc.bitcast(x, jnp.int32)
    e = jnp.right_shift(bits, 23) - 127
    m = plsc.bitcast((bits & 0x7FFFFF) | 0x3F800000, jnp.float32)
    big = m > 1.4142135
    e = e + big.astype(jnp.int32)
    m = jnp.where(big, m * 0.5, m)
    r = m - 1.0
    s = r / (r + 2.0)
    s2 = s * s
    p = 2.0 / 9.0
    for c in (2.0 / 7.0, 2.0 / 5.0, 2.0 / 3.0, 2.0):
        p = p * s2 + c
    return e.astype(jnp.float32) * LN2 + p * s


def _stirling(z):
    zi = 1.0 / z
    zi2 = zi * zi
    ser = zi * (1.0 / 12.0 + zi2 * (-1.0 / 360.0 + zi2 * (1.0 / 1260.0)))
    return (z - 0.5) * _fast_log(z) - z + HALF_LN2PI + ser


def _qb_of(wid, k):
    c = jnp.minimum(wid + NW * k, NCHUNK_G - 1)
    return jnp.minimum(c * CH, TAIL_BASE)


def _body_a(el_hbm, ci_hbm, ei_hbm, pe_hbm,
            cells_v, edges_v, out_v,
            rows0, cidx0, rows1, cidx1,
            semi0, semi1, semo, semr0, semr1):
    wid = lax.axis_index("s") * NC + lax.axis_index("c")

    for k in range(KPW):
        qb = _qb_of(wid, k)
        dst = pl.ds(k * CH, CH)
        pltpu.async_copy(ci_hbm.at[pl.ds(qb, CH)], cells_v.at[dst], semi0)
        pltpu.async_copy(ei_hbm.at[pl.ds(qb, CH)], edges_v.at[dst], semi1)
    for k in range(KPW):
        dst = pl.ds(k * CH, CH)
        pltpu.make_async_copy(ci_hbm.at[pl.ds(0, CH)], cells_v.at[dst], semi0).wait()
        pltpu.make_async_copy(ei_hbm.at[pl.ds(0, CH)], edges_v.at[dst], semi1).wait()

    bufs = ((rows0, cidx0, semr0), (rows1, cidx1, semr1))

    def stage(k, p):
        rows_v, cidx_v, semr = bufs[p]
        c0 = k * CH

        @plsc.parallel_loop(0, NG, 1, unroll=2)
        def _cidx_loop(g):
            cidx_v[pl.ds(g * L, L)] = cells_v[pl.ds(c0 + g * L, L)]
        pltpu.async_copy(el_hbm.at[cidx_v], rows_v, semr)

    def wait(p):
        rows_v, cidx_v, semr = bufs[p]
        pltpu.make_async_copy(el_hbm.at[cidx_v], rows_v, semr).wait()

    def compute(k, p):
        rows_v, cidx_v, semr = bufs[p]
        c0 = k * CH

        def one_group(q0):
            sl = pl.ds(c0 + q0, L)
            edge = edges_v[sl]
            qvec = jnp.full((L,), q0, jnp.int32) + lax.iota(jnp.int32, L)

            accs = []
            for j in range(8):
                ev = jnp.full((L,), j, jnp.int32)
                accs.append(jnp.exp(plsc.load_gather(rows_v, [qvec, ev])))
            for e in range(8, N_EDGES):
                ev = jnp.full((L,), e, jnp.int32)
                x = plsc.load_gather(rows_v, [qvec, ev])
                accs[e % 8] = accs[e % 8] + jnp.exp(x)
            while len(accs) > 1:
                accs = [accs[i] + accs[i + 1] for i in range(0, len(accs), 2)]
            s = accs[0]

            xsel = plsc.load_gather(rows_v, [qvec, edge])
            out_v[sl] = xsel - _fast_log(s)

        @plsc.parallel_loop(0, NG, 1, unroll=2)
        def _group_loop(g):
            one_group(g * L)
        pltpu.async_copy(out_v.at[pl.ds(c0, CH)], pe_hbm.at[pl.ds(_qb_of(wid, k), CH)], semo)

    stage(0, 0)

    def pair_body(i, carry):
        c = i * 2
        stage(c + 1, 1)
        wait(0)
        compute(c, 0)
        stage(c + 2, 0)
        wait(1)
        compute(c + 1, 1)
        return carry

    lax.fori_loop(0, (KPW - 1) // 2, pair_body, 0)
    wait(0)
    compute(KPW - 1, 0)
    for k in range(KPW):
        pltpu.make_async_copy(out_v.at[pl.ds(k * CH, CH)],
                              pe_hbm.at[pl.ds(0, CH)], semo).wait()


def _body_b(at_hbm, bt_hbm, t_hbm, ci_hbm, ei_hbm, pe_hbm, out_hbm,
            cells_v, edges_v, ts_v, pe_v, out_v,
            a0, b0, fidx0, a1, b1, fidx1,
            semi0, semi1, semi2, semi3, semo,
            sema0, semb0, sema1, semb1):
    wid = lax.axis_index("s") * NC + lax.axis_index("c")

    for k in range(KPW):
        qb = _qb_of(wid, k)
        dst = pl.ds(k * CH, CH)
        pltpu.async_copy(ci_hbm.at[pl.ds(qb, CH)], cells_v.at[dst], semi0)
        pltpu.async_copy(ei_hbm.at[pl.ds(qb, CH)], edges_v.at[dst], semi1)
        pltpu.async_copy(t_hbm.at[pl.ds(qb, CH)], ts_v.at[dst], semi2)
        pltpu.async_copy(pe_hbm.at[pl.ds(qb, CH)], pe_v.at[dst], semi3)
    for k in range(KPW):
        dst = pl.ds(k * CH, CH)
        pltpu.make_async_copy(ci_hbm.at[pl.ds(0, CH)], cells_v.at[dst], semi0).wait()
        pltpu.make_async_copy(ei_hbm.at[pl.ds(0, CH)], edges_v.at[dst], semi1).wait()
        pltpu.make_async_copy(t_hbm.at[pl.ds(0, CH)], ts_v.at[dst], semi2).wait()
        pltpu.make_async_copy(pe_hbm.at[pl.ds(0, CH)], pe_v.at[dst], semi3).wait()

    bufs = ((a0, b0, fidx0, sema0, semb0), (a1, b1, fidx1, sema1, semb1))

    def stage(k, p):
        a_v, b_v, fidx_v, sema, semb = bufs[p]
        c0 = k * CH

        @plsc.parallel_loop(0, NG, 1, unroll=2)
        def _fidx_loop(g):
            sl = pl.ds(c0 + g * L, L)
            fidx_v[pl.ds(g * L, L)] = edges_v[sl] * N_CELLS + cells_v[sl]
        pltpu.async_copy(at_hbm.at[fidx_v], a_v, sema)
        pltpu.async_copy(bt_hbm.at[fidx_v], b_v, semb)

    def wait(p):
        a_v, b_v, fidx_v, sema, semb = bufs[p]
        pltpu.make_async_copy(at_hbm.at[fidx_v], a_v, sema).wait()
        pltpu.make_async_copy(bt_hbm.at[fidx_v], b_v, semb).wait()

    def compute(k, p):
        a_v, b_v, fidx_v, sema, semb = bufs[p]
        c0 = k * CH

        def group_body(g):
            q0 = g * L
            sl = pl.ds(c0 + q0, L)
            tv = ts_v[sl]
            pe = pe_v[sl]
            qsl = pl.ds(q0, L)
            a = a_v[qsl]
            b = b_v[qsl]
            ab = a + b
            corr = _fast_log(a * (a + 1.0) * b * (b + 1.0) / (ab * (ab + 1.0)))
            lbc = (_stirling(a + 2.0) + _stirling(b + 2.0)
                   - _stirling(ab + 2.0) - corr)
            ltc = _fast_log(tv)
            l1tc = _fast_log(1.0 - tv)
            logp_t = (a - 1.0) * ltc + (b - 1.0) * l1tc - lbc
            out_v[sl] = pe + logp_t

        plsc.parallel_loop(0, NG, 1, unroll=2)(group_body)
        pltpu.async_copy(out_v.at[pl.ds(c0, CH)], out_hbm.at[pl.ds(_qb_of(wid, k), CH)], semo)

    stage(0, 0)

    def pair_body(i, carry):
        c = i * 2
        stage(c + 1, 1)
        wait(0)
        compute(c, 0)
        stage(c + 2, 0)
        wait(1)
        compute(c + 1, 1)
        return carry

    lax.fori_loop(0, (KPW - 1) // 2, pair_body, 0)
    wait(0)
    compute(KPW - 1, 0)
    for k in range(KPW):
        pltpu.make_async_copy(out_v.at[pl.ds(k * CH, CH)],
                              out_hbm.at[pl.ds(0, CH)], semo).wait()


_SC_PARAMS = pltpu.CompilerParams(
    needs_layout_passes=False, use_tc_tiling_on_sc=False)
_MESH = plsc.VectorSubcoreMesh(core_axis_name="c", subcore_axis_name="s",
                               num_cores=NC, num_subcores=NS)


def _make_call_a():
    return pl.kernel(
        _body_a,
        out_type=jax.ShapeDtypeStruct((B_TOTAL,), jnp.float32),
        mesh=_MESH,
        scratch_types=(
            [pltpu.VMEM((KPW * CH,), jnp.int32),
             pltpu.VMEM((KPW * CH,), jnp.int32),
             pltpu.VMEM((KPW * CH,), jnp.float32)]
            + [pltpu.VMEM((CH, N_EDGES), jnp.float32), pltpu.VMEM((CH,), jnp.int32)]
            + [pltpu.VMEM((CH, N_EDGES), jnp.float32), pltpu.VMEM((CH,), jnp.int32)]
            + 5 * [pltpu.SemaphoreType.DMA]
        ),
        compiler_params=_SC_PARAMS,
    )


def _make_call_b():
    return pl.kernel(
        _body_b,
        out_type=jax.ShapeDtypeStruct((B_TOTAL,), jnp.float32),
        mesh=_MESH,
        scratch_types=(
            [pltpu.VMEM((KPW * CH,), jnp.int32),
             pltpu.VMEM((KPW * CH,), jnp.int32),
             pltpu.VMEM((KPW * CH,), jnp.float32),
             pltpu.VMEM((KPW * CH,), jnp.float32),
             pltpu.VMEM((KPW * CH,), jnp.float32)]
            + 2 * [pltpu.VMEM((CH,), jnp.float32),
                   pltpu.VMEM((CH,), jnp.float32),
                   pltpu.VMEM((CH,), jnp.int32)]
            + 9 * [pltpu.SemaphoreType.DMA]
        ),
        compiler_params=_SC_PARAMS,
    )


@jax.jit
def _run(alpha, beta, edge_logits, t, cell_idx, edge_idx):
    ci = cell_idx.astype(jnp.int32)
    ei = edge_idx.astype(jnp.int32)
    pe = _make_call_a()(edge_logits, ci, ei)
    at_flat = alpha.T.reshape(-1)
    bt_flat = beta.T.reshape(-1)
    return _make_call_b()(at_flat, bt_flat, t, ci, ei, pe)


def kernel(alpha, beta, edge_logits, t, cell_idx, edge_idx):
    return _run(alpha, beta, edge_logits, t, cell_idx, edge_idx)


# speedup vs baseline: 1.0219x; 1.0219x over previous
import jax
import jax.numpy as jnp
from jax import lax
from jax.experimental import pallas as pl
from jax.experimental.pallas import tpu as pltpu
from jax.experimental.pallas import tpu_sc as plsc

N_CELLS = 100000
N_EDGES = 64
B_TOTAL = 100000
NC = 2
NS = 16
L = 16
NW = NC * NS
CH = 128
NG = CH // L
NCHUNK_G = -(-B_TOTAL // CH)
TAIL_BASE = B_TOTAL - CH
KPW = -(-NCHUNK_G // NW)

LN2 = 0.6931471805599453
HALF_LN2PI = 0.9189385332046727


def _fast_log(x):
    bits = plsc.bitcast(x, jnp.int32)
    e = jnp.right_shift(bits, 23) - 127
    m = plsc.bitcast((bits & 0x7FFFFF) | 0x3F800000, jnp.float32)
    big = m > 1.4142135
    e = e + big.astype(jnp.int32)
    m = jnp.where(big, m * 0.5, m)
    r = m - 1.0
    s = r / (r + 2.0)
    s2 = s * s
    p = 2.0 / 9.0
    for c in (2.0 / 7.0, 2.0 / 5.0, 2.0 / 3.0, 2.0):
        p = p * s2 + c
    return e.astype(jnp.float32) * LN2 + p * s


def _stirling(z):
    zi = 1.0 / z
    zi2 = zi * zi
    ser = zi * (1.0 / 12.0 + zi2 * (-1.0 / 360.0 + zi2 * (1.0 / 1260.0)))
    return (z - 0.5) * _fast_log(z) - z + HALF_LN2PI + ser


def _qb_of(wid, k):
    c = jnp.minimum(wid + NW * k, NCHUNK_G - 1)
    return jnp.minimum(c * CH, TAIL_BASE)


def _body_a(el_hbm, ci_hbm, ei_hbm, pe_hbm,
            cells_v, edges_v, out_v,
            rows0, cidx0, rows1, cidx1,
            semi0, semi1, semo, semr0, semr1):
    wid = lax.axis_index("s") * NC + lax.axis_index("c")

    for k in range(KPW):
        qb = _qb_of(wid, k)
        dst = pl.ds(k * CH, CH)
        pltpu.async_copy(ci_hbm.at[pl.ds(qb, CH)], cells_v.at[dst], semi0)
        pltpu.async_copy(ei_hbm.at[pl.ds(qb, CH)], edges_v.at[dst], semi1)
    for k in range(KPW):
        dst = pl.ds(k * CH, CH)
        pltpu.make_async_copy(ci_hbm.at[pl.ds(0, CH)], cells_v.at[dst], semi0).wait()
        pltpu.make_async_copy(ei_hbm.at[pl.ds(0, CH)], edges_v.at[dst], semi1).wait()

    bufs = ((rows0, cidx0, semr0), (rows1, cidx1, semr1))

    def stage(k, p):
        rows_v, cidx_v, semr = bufs[p]
        c0 = k * CH

        def cidx_body(g, carry2):
            cidx_v[pl.ds(g * L, L)] = cells_v[pl.ds(c0 + g * L, L)]
            return carry2

        lax.fori_loop(0, NG, cidx_body, 0)
        pltpu.async_copy(el_hbm.at[cidx_v], rows_v, semr)

    def wait(p):
        rows_v, cidx_v, semr = bufs[p]
        pltpu.make_async_copy(el_hbm.at[cidx_v], rows_v, semr).wait()

    def compute(k, p):
        rows_v, cidx_v, semr = bufs[p]
        c0 = k * CH

        def one_group(q0):
            sl = pl.ds(c0 + q0, L)
            edge = edges_v[sl]
            qvec = jnp.full((L,), q0, jnp.int32) + lax.iota(jnp.int32, L)

            accs = []
            for j in range(8):
                ev = jnp.full((L,), j, jnp.int32)
                accs.append(jnp.exp(plsc.load_gather(rows_v, [qvec, ev])))
            for e in range(8, N_EDGES):
                ev = jnp.full((L,), e, jnp.int32)
                x = plsc.load_gather(rows_v, [qvec, ev])
                accs[e % 8] = accs[e % 8] + jnp.exp(x)
            while len(accs) > 1:
                accs = [accs[i] + accs[i + 1] for i in range(0, len(accs), 2)]
            s = accs[0]

            xsel = plsc.load_gather(rows_v, [qvec, edge])
            out_v[sl] = xsel - _fast_log(s)

        @plsc.parallel_loop(0, NG, 1, unroll=2)
        def _group_loop(g):
            one_group(g * L)
        pltpu.async_copy(out_v.at[pl.ds(c0, CH)], pe_hbm.at[pl.ds(_qb_of(wid, k), CH)], semo)

    stage(0, 0)

    def pair_body(i, carry):
        c = i * 2
        stage(c + 1, 1)
        wait(0)
        compute(c, 0)
        stage(c + 2, 0)
        wait(1)
        compute(c + 1, 1)
        return carry

    lax.fori_loop(0, (KPW - 1) // 2, pair_body, 0)
    wait(0)
    compute(KPW - 1, 0)
    for k in range(KPW):
        pltpu.make_async_copy(out_v.at[pl.ds(k * CH, CH)],
                              pe_hbm.at[pl.ds(0, CH)], semo).wait()


def _body_b(at_hbm, bt_hbm, t_hbm, ci_hbm, ei_hbm, pe_hbm, out_hbm,
            cells_v, edges_v, ts_v, pe_v, out_v,
            a0, b0, fidx0, a1, b1, fidx1,
            semi0, semi1, semi2, semi3, semo,
            sema0, semb0, sema1, semb1):
    wid = lax.axis_index("s") * NC + lax.axis_index("c")

    for k in range(KPW):
        qb = _qb_of(wid, k)
        dst = pl.ds(k * CH, CH)
        pltpu.async_copy(ci_hbm.at[pl.ds(qb, CH)], cells_v.at[dst], semi0)
        pltpu.async_copy(ei_hbm.at[pl.ds(qb, CH)], edges_v.at[dst], semi1)
        pltpu.async_copy(t_hbm.at[pl.ds(qb, CH)], ts_v.at[dst], semi2)
        pltpu.async_copy(pe_hbm.at[pl.ds(qb, CH)], pe_v.at[dst], semi3)
    for k in range(KPW):
        dst = pl.ds(k * CH, CH)
        pltpu.make_async_copy(ci_hbm.at[pl.ds(0, CH)], cells_v.at[dst], semi0).wait()
        pltpu.make_async_copy(ei_hbm.at[pl.ds(0, CH)], edges_v.at[dst], semi1).wait()
        pltpu.make_async_copy(t_hbm.at[pl.ds(0, CH)], ts_v.at[dst], semi2).wait()
        pltpu.make_async_copy(pe_hbm.at[pl.ds(0, CH)], pe_v.at[dst], semi3).wait()

    bufs = ((a0, b0, fidx0, sema0, semb0), (a1, b1, fidx1, sema1, semb1))

    def stage(k, p):
        a_v, b_v, fidx_v, sema, semb = bufs[p]
        c0 = k * CH

        def fidx_body(g, carry2):
            sl = pl.ds(c0 + g * L, L)
            fidx_v[pl.ds(g * L, L)] = edges_v[sl] * N_CELLS + cells_v[sl]
            return carry2

        lax.fori_loop(0, NG, fidx_body, 0)
        pltpu.async_copy(at_hbm.at[fidx_v], a_v, sema)
        pltpu.async_copy(bt_hbm.at[fidx_v], b_v, semb)

    def wait(p):
        a_v, b_v, fidx_v, sema, semb = bufs[p]
        pltpu.make_async_copy(at_hbm.at[fidx_v], a_v, sema).wait()
        pltpu.make_async_copy(bt_hbm.at[fidx_v], b_v, semb).wait()

    def compute(k, p):
        a_v, b_v, fidx_v, sema, semb = bufs[p]
        c0 = k * CH

        def group_body(g, carry2):
            q0 = g * L
            sl = pl.ds(c0 + q0, L)
            tv = ts_v[sl]
            pe = pe_v[sl]
            qsl = pl.ds(q0, L)
            a = a_v[qsl]
            b = b_v[qsl]
            ab = a + b
            corr = _fast_log(a * (a + 1.0) * b * (b + 1.0) / (ab * (ab + 1.0)))
            lbc = (_stirling(a + 2.0) + _stirling(b + 2.0)
                   - _stirling(ab + 2.0) - corr)
            ltc = _fast_log(tv)
            l1tc = _fast_log(1.0 - tv)
            logp_t = (a - 1.0) * ltc + (b - 1.0) * l1tc - lbc
            out_v[sl] = pe + logp_t
            return carry2

        lax.fori_loop(0, NG, group_body, 0)
        pltpu.async_copy(out_v.at[pl.ds(c0, CH)], out_hbm.at[pl.ds(_qb_of(wid, k), CH)], semo)

    stage(0, 0)

    def pair_body(i, carry):
        c = i * 2
        stage(c + 1, 1)
        wait(0)
        compute(c, 0)
        stage(c + 2, 0)
        wait(1)
        compute(c + 1, 1)
        return carry

    lax.fori_loop(0, (KPW - 1) // 2, pair_body, 0)
    wait(0)
    compute(KPW - 1, 0)
    for k in range(KPW):
        pltpu.make_async_copy(out_v.at[pl.ds(k * CH, CH)],
                              out_hbm.at[pl.ds(0, CH)], semo).wait()


_SC_PARAMS = pltpu.CompilerParams(
    needs_layout_passes=False, use_tc_tiling_on_sc=False)
_MESH = plsc.VectorSubcoreMesh(core_axis_name="c", subcore_axis_name="s",
                               num_cores=NC, num_subcores=NS)


def _make_call_a():
    return pl.kernel(
        _body_a,
        out_type=jax.ShapeDtypeStruct((B_TOTAL,), jnp.float32),
        mesh=_MESH,
        scratch_types=(
            [pltpu.VMEM((KPW * CH,), jnp.int32),
             pltpu.VMEM((KPW * CH,), jnp.int32),
             pltpu.VMEM((KPW * CH,), jnp.float32)]
            + [pltpu.VMEM((CH, N_EDGES), jnp.float32), pltpu.VMEM((CH,), jnp.int32)]
            + [pltpu.VMEM((CH, N_EDGES), jnp.float32), pltpu.VMEM((CH,), jnp.int32)]
            + 5 * [pltpu.SemaphoreType.DMA]
        ),
        compiler_params=_SC_PARAMS,
    )


def _make_call_b():
    return pl.kernel(
        _body_b,
        out_type=jax.ShapeDtypeStruct((B_TOTAL,), jnp.float32),
        mesh=_MESH,
        scratch_types=(
            [pltpu.VMEM((KPW * CH,), jnp.int32),
             pltpu.VMEM((KPW * CH,), jnp.int32),
             pltpu.VMEM((KPW * CH,), jnp.float32),
             pltpu.VMEM((KPW * CH,), jnp.float32),
             pltpu.VMEM((KPW * CH,), jnp.float32)]
            + 2 * [pltpu.VMEM((CH,), jnp.float32),
                   pltpu.VMEM((CH,), jnp.float32),
                   pltpu.VMEM((CH,), jnp.int32)]
            + 9 * [pltpu.SemaphoreType.DMA]
        ),
        compiler_params=_SC_PARAMS,
    )


@jax.jit
def _run(alpha, beta, edge_logits, t, cell_idx, edge_idx):
    ci = cell_idx.astype(jnp.int32)
    ei = edge_idx.astype(jnp.int32)
    pe = _make_call_a()(edge_logits, ci, ei)
    at_flat = alpha.T.reshape(-1)
    bt_flat = beta.T.reshape(-1)
    return _make_call_b()(at_flat, bt_flat, t, ci, ei, pe)


def kernel(alpha, beta, edge_logits, t, cell_idx, edge_idx):
    return _run(alpha, beta, edge_logits, t, cell_idx, edge_idx)
